# baseline (speedup 1.0000x reference)
import functools

import jax
import jax.numpy as jnp
from jax import lax
from jax.experimental import pallas as pl
from jax.experimental.pallas import tpu as pltpu
from jax.experimental.pallas import tpu_sc as plsc

NC = 2
NS = 16
NW = NC * NS
CHUNK = 128
NBUF = 4


def _mesh():
    return plsc.VectorSubcoreMesh(core_axis_name="c", subcore_axis_name="s")


def _make_degree_kernel(n_pad, k):
    n2 = 2 * n_pad
    zchunk = n2 // 16
    assert n2 % zchunk == 0 and zchunk % 16 == 0

    @functools.partial(
        pl.kernel,
        out_type=jax.ShapeDtypeStruct((NC, n2), jnp.float32),
        mesh=_mesh(),
        scratch_types=[
            pltpu.VMEM((k, CHUNK), jnp.int32),
            pltpu.VMEM((k, CHUNK), jnp.int32),
            pltpu.VMEM((2 * k, CHUNK), jnp.int32),
            pltpu.VMEM((zchunk,), jnp.float32),
            pltpu.VMEM((CHUNK,), jnp.float32),
            pltpu.VMEM_SHARED((n2,), jnp.float32),
            pltpu.SemaphoreType.DMA,
        ],
        compiler_params=pltpu.CompilerParams(use_tc_tiling_on_sc=False),
    )
    def deg_kernel(src_hbm, dst_hbm, out_hbm,
                   src_v, dst_v, didx_v, zbuf, ones_v, acc, sem):
        cid = lax.axis_index("c")
        sid = lax.axis_index("s")
        wid = sid * NC + cid

        @pl.when(sid == 0)
        def _():
            def zfill(i, carry):
                zbuf[pl.ds(i * 16, 16)] = jnp.zeros((16,), jnp.float32)
                return carry
            lax.fori_loop(0, zchunk // 16, zfill, 0)
            for i in range(n2 // zchunk):
                pltpu.sync_copy(zbuf, acc.at[pl.ds(i * zchunk, zchunk)])

        def ofill(i, carry):
            ones_v[pl.ds(i * 16, 16)] = jnp.ones((16,), jnp.float32)
            return carry
        lax.fori_loop(0, CHUNK // 16, ofill, 0)

        pltpu.sync_copy(src_hbm.at[wid], src_v)
        pltpu.sync_copy(dst_hbm.at[wid], dst_v)

        def build(j, carry):
            for t in range(CHUNK // 16):
                sl = pl.ds(t * 16, 16)
                didx_v[j, sl] = src_v[j, sl] * 2
                didx_v[j + k, sl] = dst_v[j, sl] * 2 + 1
            return carry
        lax.fori_loop(0, k, build, 0)
        plsc.subcore_barrier()

        def fire(j, carry):
            pltpu.async_copy(ones_v, acc.at[didx_v.at[j]], sem, add=True)
            return carry
        lax.fori_loop(0, 2 * k, fire, 0)

        def drain(j, carry):
            pltpu.make_async_copy(ones_v, acc.at[didx_v.at[j]], sem).wait()
            return carry
        lax.fori_loop(0, 2 * k, drain, 0)
        plsc.subcore_barrier()

        @pl.when(sid == 0)
        def _():
            pltpu.sync_copy(acc, out_hbm.at[cid])

    return deg_kernel


def _make_agg_kernel(n_pad, k, f):
    rows_per_tile = n_pad // NS
    assert rows_per_tile % 8 == 0

    @functools.partial(
        pl.kernel,
        out_type=jax.ShapeDtypeStruct((NC, n_pad, f), jnp.float32),
        mesh=_mesh(),
        scratch_types=[
            pltpu.VMEM((k, CHUNK), jnp.int32),
            pltpu.VMEM((k, CHUNK), jnp.int32),
            pltpu.VMEM((NBUF, CHUNK, f), jnp.float32),
            pltpu.VMEM((rows_per_tile, f), jnp.float32),
            pltpu.VMEM_SHARED((n_pad, f), jnp.float32),
            pltpu.SemaphoreType.DMA((NBUF,)),
            pltpu.SemaphoreType.DMA((NBUF,)),
        ],
        compiler_params=pltpu.CompilerParams(use_tc_tiling_on_sc=False),
    )
    def agg_kernel(h_hbm, src_hbm, dst_hbm, out_hbm,
                   src_v, dst_v, buf, zbuf, acc, gsem, ssem):
        cid = lax.axis_index("c")
        sid = lax.axis_index("s")
        wid = sid * NC + cid

        def zfill(i, carry):
            zbuf[i, :] = jnp.zeros((f,), jnp.float32)
            return carry
        lax.fori_loop(0, rows_per_tile, zfill, 0)
        pltpu.sync_copy(zbuf, acc.at[pl.ds(sid * rows_per_tile, rows_per_tile)])

        pltpu.sync_copy(src_hbm.at[wid], src_v)
        pltpu.sync_copy(dst_hbm.at[wid], dst_v)
        plsc.subcore_barrier()

        for b in range(2):
            pltpu.async_copy(h_hbm.at[src_v.at[b]], buf.at[b], gsem.at[b])

        def body(j, carry):
            slot = lax.rem(j, NBUF)
            pltpu.make_async_copy(h_hbm.at[src_v.at[j]],
                                  buf.at[slot], gsem.at[slot]).wait()
            pltpu.async_copy(buf.at[slot], acc.at[dst_v.at[j]],
                             ssem.at[slot], add=True)

            @pl.when(j + 2 < k)
            def _():
                ns = lax.rem(j + 2, NBUF)

                @pl.when(j >= 2)
                def _():
                    pltpu.make_async_copy(buf.at[ns],
                                          acc.at[dst_v.at[j - 2]],
                                          ssem.at[ns]).wait()
                pltpu.async_copy(h_hbm.at[src_v.at[j + 2]],
                                 buf.at[ns], gsem.at[ns])
            return carry

        lax.fori_loop(0, k, body, 0)
        for t in range(min(NBUF, k)):
            jj = k - min(NBUF, k) + t
            pltpu.make_async_copy(buf.at[jj % NBUF],
                                  acc.at[dst_v.at[jj]],
                                  ssem.at[jj % NBUF]).wait()
        plsc.subcore_barrier()

        pltpu.sync_copy(
            acc.at[pl.ds(sid * rows_per_tile, rows_per_tile)],
            out_hbm.at[cid, pl.ds(sid * rows_per_tile, rows_per_tile)])

    return agg_kernel


def _tc_matmul(x, w, bm=2000):
    n, kin = x.shape
    f = w.shape[1]

    def mm_kernel(x_ref, w_ref, o_ref):
        o_ref[...] = lax.dot_general(
            x_ref[...], w_ref[...], (((1,), (0,)), ((), ())),
            preferred_element_type=jnp.float32)

    return pl.pallas_call(
        mm_kernel,
        grid=(n // bm,),
        in_specs=[pl.BlockSpec((bm, kin), lambda i: (i, 0)),
                  pl.BlockSpec((kin, f), lambda i: (0, 0))],
        out_specs=pl.BlockSpec((bm, f), lambda i: (i, 0)),
        out_shape=jax.ShapeDtypeStruct((n, f), jnp.float32),
    )(x, w)


def _tc_scale_by_dsqo(y, deg, n_pad):
    n, f = y.shape

    def body(y_ref, d_ref, o_ref):
        d = d_ref[0, :n, :] + d_ref[1, :n, :]
        dsqo = lax.rsqrt(jnp.maximum(d[:, 0:1], 1.0))
        o_ref[:n, :] = y_ref[...] * dsqo
        o_ref[n:, :] = jnp.zeros((n_pad - n, f), jnp.float32)

    return pl.pallas_call(
        body,
        out_shape=jax.ShapeDtypeStruct((n_pad, f), jnp.float32),
    )(y, deg)


def _tc_relu_rescale(agg_parts, deg, b1, n):
    _, n_pad, f = agg_parts.shape

    def body(a_ref, d_ref, b_ref, o_ref):
        a = a_ref[0, :n, :] + a_ref[1, :n, :]
        d = d_ref[0, :n, :] + d_ref[1, :n, :]
        dsqo = lax.rsqrt(jnp.maximum(d[:, 0:1], 1.0))
        dsqi = lax.rsqrt(jnp.maximum(d[:, 1:2], 1.0))
        h = jnp.maximum(a * dsqi + b_ref[...], 0.0)
        o_ref[:n, :] = h * dsqo
        o_ref[n:, :] = jnp.zeros((n_pad - n, f), jnp.float32)

    return pl.pallas_call(
        body,
        out_shape=jax.ShapeDtypeStruct((n_pad, f), jnp.float32),
    )(agg_parts, deg, b1.reshape(1, f))


def _tc_final(agg_parts, deg, w2, b2, n):
    _, n_pad, f = agg_parts.shape
    fo = w2.shape[1]

    def body(a_ref, d_ref, w_ref, b_ref, o_ref):
        a = a_ref[0, :n, :] + a_ref[1, :n, :]
        d = d_ref[0, :n, :] + d_ref[1, :n, :]
        dsqi = lax.rsqrt(jnp.maximum(d[:, 1:2], 1.0))
        h = a * dsqi
        o_ref[...] = lax.dot_general(
            h, w_ref[...], (((1,), (0,)), ((), ())),
            preferred_element_type=jnp.float32) + b_ref[...]

    return pl.pallas_call(
        body,
        out_shape=jax.ShapeDtypeStruct((n, fo), jnp.float32),
    )(agg_parts, deg, w2, b2.reshape(1, fo))


def kernel(features, edge_index, W1, b1, W2, b2):
    n, _ = features.shape
    e = edge_index.shape[1]
    n_pad = ((n + 8 * NS - 1) // (8 * NS)) * (8 * NS)
    trash = n_pad - 1

    k = -(-e // (NW * CHUNK))
    e_pad = NW * k * CHUNK
    pad = jnp.full((e_pad - e,), trash, jnp.int32)
    src_r = jnp.concatenate([edge_index[0], pad]).reshape(NW, k, CHUNK)
    dst_r = jnp.concatenate([edge_index[1], pad]).reshape(NW, k, CHUNK)

    deg = _make_degree_kernel(n_pad, k)(src_r, dst_r).reshape(NC, n_pad, 2)
    y = _tc_matmul(features, W1)
    h1 = _tc_scale_by_dsqo(y, deg, n_pad)

    agg16 = _make_agg_kernel(n_pad, k, 16)
    a1 = agg16(h1, src_r, dst_r)
    scaled = _tc_relu_rescale(a1, deg, b1, n)
    a2 = agg16(scaled, src_r, dst_r)
    return _tc_final(a2, deg, W2, b2, n)
